# baseline (speedup 1.0000x reference)
import functools

import jax
import jax.numpy as jnp
from jax import lax
from jax.experimental import pallas as pl
from jax.experimental.pallas import tpu as pltpu
from jax.experimental.pallas import tpu_sc as plsc

B, BANDS, N, E = 8, 9, 512, 16384
G = B * BANDS
IN_C, HID, OUT = 32, 64, 128
NW = 32
Q = 4
EQ = E // Q
TASKS = G * Q
TPW = TASKS // NW
L = 16
NC = N * IN_C


def _sc_scatter(x, ei, ea, params):
  mesh = plsc.VectorSubcoreMesh(core_axis_name="c", subcore_axis_name="s")

  @functools.partial(
      pl.kernel,
      mesh=mesh,
      compiler_params=pltpu.CompilerParams(use_tc_tiling_on_sc=False),
      out_type=jax.ShapeDtypeStruct((G, Q, NC), jnp.float32),
      scratch_types=[
          pltpu.VMEM((2, NC), jnp.float32),
          pltpu.VMEM((NC,), jnp.float32),
          pltpu.VMEM((2, EQ), jnp.int32),
          pltpu.VMEM((2, EQ), jnp.int32),
          pltpu.VMEM((2, EQ), jnp.float32),
          pltpu.VMEM((4 * L,), jnp.float32),
          pltpu.SemaphoreType.DMA((2,)),
      ],
  )
  def k(x_hbm, ei_hbm, ea_hbm, par_hbm, out_hbm,
        x_v, aggr_v, src_v, dst_v, ea_v, par_v, sem):
    wid = lax.axis_index("s") * 2 + lax.axis_index("c")
    pltpu.sync_copy(par_hbm, par_v)
    we0 = par_v[pl.ds(0, L)]
    we1 = par_v[pl.ds(L, L)]
    be0 = par_v[pl.ds(2 * L, L)]
    be1 = par_v[pl.ds(3 * L, L)]
    zero = jnp.zeros((L,), jnp.float32)

    def in_copies(t, b):
      g = t // Q
      q = t % Q
      return (
          pltpu.make_async_copy(x_hbm.at[g, 0], x_v.at[b], sem.at[b]),
          pltpu.make_async_copy(
              ei_hbm.at[g, 0, pl.ds(q * EQ, EQ)], src_v.at[b], sem.at[b]),
          pltpu.make_async_copy(
              ei_hbm.at[g, 1, pl.ds(q * EQ, EQ)], dst_v.at[b], sem.at[b]),
          pltpu.make_async_copy(
              ea_hbm.at[g, 0, pl.ds(q * EQ, EQ)], ea_v.at[b], sem.at[b]),
      )

    def start_in(t, b):
      for c in in_copies(t, b):
        c.start()

    def wait_in(t, b):
      for c in in_copies(t, b):
        c.wait()

    def compute(t, b):
      g = t // Q
      q = t % Q

      @plsc.parallel_loop(0, NC // L, unroll=8)
      def zloop(n):
        aggr_v[pl.ds(n * L, L)] = zero

      xb = x_v.at[b]
      sb = src_v.at[b]
      db = dst_v.at[b]
      ab = ea_v.at[b]

      @plsc.parallel_loop(0, N, unroll=4)
      def preadd(n):
        plsc.addupdate(xb.at[pl.ds(n * IN_C, L)], be0)
        plsc.addupdate(xb.at[pl.ds(n * IN_C + L, L)], be1)

      @plsc.parallel_loop(0, EQ // L, unroll=4)
      def eloop(j):
        base = j * L
        s16 = sb[pl.ds(base, L)] * IN_C
        d16 = db[pl.ds(base, L)] * IN_C
        a16 = ab[pl.ds(base, L)]
        for lane in range(L):
          s = pl.multiple_of(s16[lane], IN_C)
          d = pl.multiple_of(d16[lane], IN_C)
          a = a16[lane]
          m0 = jnp.maximum(xb[pl.ds(s, L)] + a * we0, 0.0)
          m1 = jnp.maximum(xb[pl.ds(s + L, L)] + a * we1, 0.0)
          plsc.addupdate(aggr_v.at[pl.ds(d, L)], m0)
          plsc.addupdate(aggr_v.at[pl.ds(d + L, L)], m1)

      pltpu.sync_copy(aggr_v, out_hbm.at[g, q])

    t0 = wid * TPW
    start_in(t0, 0)

    def pair(i, carry):
      ta = t0 + 2 * i
      start_in(ta + 1, 1)
      wait_in(ta, 0)
      compute(ta, 0)
      start_in(ta + 2, 0)
      wait_in(ta + 1, 1)
      compute(ta + 1, 1)
      return carry

    lax.fori_loop(0, (TPW - 1) // 2, pair, 0)
    tl = t0 + TPW - 1
    wait_in(tl, 0)
    compute(tl, 0)

  return k(x, ei, ea, params)


GB = 8


RPG = NC // 128


def _mlp_body(x_ref, p_ref, w1b_ref, b1t_ref, sel_ref, w2_ref, b2_ref,
              o_ref):
  p = p_ref[...]
  psum = (p[:, 0 * RPG:1 * RPG] + p[:, 1 * RPG:2 * RPG]
          + p[:, 2 * RPG:3 * RPG] + p[:, 3 * RPG:4 * RPG])
  h = x_ref[...] + psum
  a1 = jnp.maximum(
      h.reshape(GB * RPG, 128) @ w1b_ref[...] + b1t_ref[...], 0.0)
  s = a1.reshape(GB, RPG, 4 * HID).sum(axis=1)
  pooled = (s @ sel_ref[...]) * (1.0 / N)
  o_ref[...] = pooled @ w2_ref[...] + b2_ref[...]


def _mlp(x, partials, W1blk, b1t, sel, W2, b2):
  return pl.pallas_call(
      _mlp_body,
      out_shape=jax.ShapeDtypeStruct((G, OUT), jnp.float32),
      grid=(G // GB,),
      in_specs=[
          pl.BlockSpec((GB, RPG, 128), lambda i: (i, 0, 0)),
          pl.BlockSpec((GB, Q * RPG, 128), lambda i: (i, 0, 0)),
          pl.BlockSpec((128, 4 * HID), lambda i: (0, 0)),
          pl.BlockSpec((1, 4 * HID), lambda i: (0, 0)),
          pl.BlockSpec((4 * HID, HID), lambda i: (0, 0)),
          pl.BlockSpec((HID, OUT), lambda i: (0, 0)),
          pl.BlockSpec((1, OUT), lambda i: (0, 0)),
      ],
      out_specs=pl.BlockSpec((GB, OUT), lambda i: (i, 0)),
  )(x, partials, W1blk, b1t, sel, W2, b2)


def kernel(x, edge_index, edge_attr, W_edge, b_edge, W1, b1, W2, b2):
  x3 = x.reshape(G, N, IN_C)
  ei = edge_index.reshape(G, 2, E).astype(jnp.int32)
  ea = edge_attr.reshape(G, 1, E)
  params = jnp.concatenate(
      [W_edge.reshape(-1), b_edge.reshape(-1)]).astype(jnp.float32)
  partials = _sc_scatter(x3.reshape(G, 1, NC), ei, ea, params)
  W1blk = jnp.kron(jnp.eye(4, dtype=W1.dtype), W1)
  b1t = jnp.tile(b1, 4).reshape(1, 4 * HID)
  sel = jnp.tile(jnp.eye(HID, dtype=W1.dtype), (4, 1))
  tokens = _mlp(x3.reshape(G, RPG, 128),
                partials.reshape(G, Q * RPG, 128),
                W1blk, b1t, sel, W2, b2.reshape(1, OUT))
  return tokens.reshape(B, BANDS, OUT)


# speedup vs baseline: 1.0458x; 1.0458x over previous
import functools

import jax
import jax.numpy as jnp
from jax import lax
from jax.experimental import pallas as pl
from jax.experimental.pallas import tpu as pltpu
from jax.experimental.pallas import tpu_sc as plsc

B, BANDS, N, E = 8, 9, 512, 16384
G = B * BANDS
IN_C, HID, OUT = 32, 64, 128
NW = 32
Q = 4
EQ = E // Q
TASKS = G * Q
TPW = TASKS // NW
L = 16
NC = N * IN_C


def _sc_scatter(x, ei, ea, params):
  mesh = plsc.VectorSubcoreMesh(core_axis_name="c", subcore_axis_name="s")

  @functools.partial(
      pl.kernel,
      mesh=mesh,
      compiler_params=pltpu.CompilerParams(use_tc_tiling_on_sc=False),
      out_type=jax.ShapeDtypeStruct((G, Q, NC), jnp.float32),
      scratch_types=[
          pltpu.VMEM((2, NC), jnp.float32),
          pltpu.VMEM((NC,), jnp.float32),
          pltpu.VMEM((2, EQ), jnp.int32),
          pltpu.VMEM((2, EQ), jnp.int32),
          pltpu.VMEM((2, EQ), jnp.float32),
          pltpu.VMEM((4 * L,), jnp.float32),
          pltpu.SemaphoreType.DMA((2,)),
      ],
  )
  def k(x_hbm, ei_hbm, ea_hbm, par_hbm, out_hbm,
        x_v, aggr_v, src_v, dst_v, ea_v, par_v, sem):
    wid = lax.axis_index("s") * 2 + lax.axis_index("c")
    pltpu.sync_copy(par_hbm, par_v)
    we0 = par_v[pl.ds(0, L)]
    we1 = par_v[pl.ds(L, L)]
    be0 = par_v[pl.ds(2 * L, L)]
    be1 = par_v[pl.ds(3 * L, L)]
    zero = jnp.zeros((L,), jnp.float32)

    def in_copies(t, b):
      g = t // Q
      q = t % Q
      return (
          pltpu.make_async_copy(x_hbm.at[g, 0], x_v.at[b], sem.at[b]),
          pltpu.make_async_copy(
              ei_hbm.at[g, 0, pl.ds(q * EQ, EQ)], src_v.at[b], sem.at[b]),
          pltpu.make_async_copy(
              ei_hbm.at[g, 1, pl.ds(q * EQ, EQ)], dst_v.at[b], sem.at[b]),
          pltpu.make_async_copy(
              ea_hbm.at[g, 0, pl.ds(q * EQ, EQ)], ea_v.at[b], sem.at[b]),
      )

    def start_in(t, b):
      for c in in_copies(t, b):
        c.start()

    def wait_in(t, b):
      for c in in_copies(t, b):
        c.wait()

    def compute(t, b):
      g = t // Q
      q = t % Q

      @plsc.parallel_loop(0, NC // L, unroll=8)
      def zloop(n):
        aggr_v[pl.ds(n * L, L)] = zero

      xb = x_v.at[b]
      sb = src_v.at[b]
      db = dst_v.at[b]
      ab = ea_v.at[b]

      @plsc.parallel_loop(0, N, unroll=4)
      def preadd(n):
        plsc.addupdate(xb.at[pl.ds(n * IN_C, L)], be0)
        plsc.addupdate(xb.at[pl.ds(n * IN_C + L, L)], be1)

      @plsc.parallel_loop(0, EQ // L, unroll=3)
      def eloop(j):
        base = j * L
        s16 = sb[pl.ds(base, L)] * IN_C
        d16 = db[pl.ds(base, L)] * IN_C
        a16 = ab[pl.ds(base, L)]
        for lane in range(L):
          s = pl.multiple_of(s16[lane], IN_C)
          d = pl.multiple_of(d16[lane], IN_C)
          a = a16[lane]
          m0 = jnp.maximum(xb[pl.ds(s, L)] + a * we0, 0.0)
          m1 = jnp.maximum(xb[pl.ds(s + L, L)] + a * we1, 0.0)
          plsc.addupdate(aggr_v.at[pl.ds(d, L)], m0)
          plsc.addupdate(aggr_v.at[pl.ds(d + L, L)], m1)

      pltpu.sync_copy(aggr_v, out_hbm.at[g, q])

    t0 = wid * TPW
    start_in(t0, 0)

    def pair(i, carry):
      ta = t0 + 2 * i
      start_in(ta + 1, 1)
      wait_in(ta, 0)
      compute(ta, 0)
      start_in(ta + 2, 0)
      wait_in(ta + 1, 1)
      compute(ta + 1, 1)
      return carry

    lax.fori_loop(0, (TPW - 1) // 2, pair, 0)
    tl = t0 + TPW - 1
    wait_in(tl, 0)
    compute(tl, 0)

  return k(x, ei, ea, params)


GB = 8


RPG = NC // 128


def _mlp_body(x_ref, p_ref, w1b_ref, b1t_ref, sel_ref, w2_ref, b2_ref,
              o_ref):
  p = p_ref[...]
  psum = (p[:, 0 * RPG:1 * RPG] + p[:, 1 * RPG:2 * RPG]
          + p[:, 2 * RPG:3 * RPG] + p[:, 3 * RPG:4 * RPG])
  h = x_ref[...] + psum
  a1 = jnp.maximum(
      h.reshape(GB * RPG, 128) @ w1b_ref[...] + b1t_ref[...], 0.0)
  s = a1.reshape(GB, RPG, 4 * HID).sum(axis=1)
  pooled = (s @ sel_ref[...]) * (1.0 / N)
  o_ref[...] = pooled @ w2_ref[...] + b2_ref[...]


def _mlp(x, partials, W1blk, b1t, sel, W2, b2):
  return pl.pallas_call(
      _mlp_body,
      out_shape=jax.ShapeDtypeStruct((G, OUT), jnp.float32),
      grid=(G // GB,),
      in_specs=[
          pl.BlockSpec((GB, RPG, 128), lambda i: (i, 0, 0)),
          pl.BlockSpec((GB, Q * RPG, 128), lambda i: (i, 0, 0)),
          pl.BlockSpec((128, 4 * HID), lambda i: (0, 0)),
          pl.BlockSpec((1, 4 * HID), lambda i: (0, 0)),
          pl.BlockSpec((4 * HID, HID), lambda i: (0, 0)),
          pl.BlockSpec((HID, OUT), lambda i: (0, 0)),
          pl.BlockSpec((1, OUT), lambda i: (0, 0)),
      ],
      out_specs=pl.BlockSpec((GB, OUT), lambda i: (i, 0)),
  )(x, partials, W1blk, b1t, sel, W2, b2)


def kernel(x, edge_index, edge_attr, W_edge, b_edge, W1, b1, W2, b2):
  x3 = x.reshape(G, N, IN_C)
  ei = edge_index.reshape(G, 2, E).astype(jnp.int32)
  ea = edge_attr.reshape(G, 1, E)
  params = jnp.concatenate(
      [W_edge.reshape(-1), b_edge.reshape(-1)]).astype(jnp.float32)
  partials = _sc_scatter(x3.reshape(G, 1, NC), ei, ea, params)
  W1blk = jnp.kron(jnp.eye(4, dtype=W1.dtype), W1)
  b1t = jnp.tile(b1, 4).reshape(1, 4 * HID)
  sel = jnp.tile(jnp.eye(HID, dtype=W1.dtype), (4, 1))
  tokens = _mlp(x3.reshape(G, RPG, 128),
                partials.reshape(G, Q * RPG, 128),
                W1blk, b1t, sel, W2, b2.reshape(1, OUT))
  return tokens.reshape(B, BANDS, OUT)


# speedup vs baseline: 1.0649x; 1.0182x over previous
import functools

import jax
import jax.numpy as jnp
from jax import lax
from jax.experimental import pallas as pl
from jax.experimental.pallas import tpu as pltpu
from jax.experimental.pallas import tpu_sc as plsc

B, BANDS, N, E = 8, 9, 512, 16384
G = B * BANDS
IN_C, HID, OUT = 32, 64, 128
NW = 32
Q = 4
EQ = E // Q
TASKS = G * Q
TPW = TASKS // NW
L = 16
NC = N * IN_C


def _sc_scatter(x, ei, ea, params):
  mesh = plsc.VectorSubcoreMesh(core_axis_name="c", subcore_axis_name="s")

  @functools.partial(
      pl.kernel,
      mesh=mesh,
      compiler_params=pltpu.CompilerParams(use_tc_tiling_on_sc=False),
      out_type=jax.ShapeDtypeStruct((G, Q, NC), jnp.float32),
      scratch_types=[
          pltpu.VMEM((2, NC), jnp.float32),
          pltpu.VMEM((NC,), jnp.float32),
          pltpu.VMEM((2, EQ), jnp.int32),
          pltpu.VMEM((2, EQ), jnp.int32),
          pltpu.VMEM((2, EQ), jnp.float32),
          pltpu.VMEM((4 * L,), jnp.float32),
          pltpu.SemaphoreType.DMA((2,)),
      ],
  )
  def k(x_hbm, ei_hbm, ea_hbm, par_hbm, out_hbm,
        x_v, aggr_v, src_v, dst_v, ea_v, par_v, sem):
    wid = lax.axis_index("s") * 2 + lax.axis_index("c")
    pltpu.sync_copy(par_hbm, par_v)
    we0 = par_v[pl.ds(0, L)]
    we1 = par_v[pl.ds(L, L)]
    be0 = par_v[pl.ds(2 * L, L)]
    be1 = par_v[pl.ds(3 * L, L)]
    zero = jnp.zeros((L,), jnp.float32)

    def in_copies(t, b):
      g = t // Q
      q = t % Q
      return (
          pltpu.make_async_copy(x_hbm.at[g, 0], x_v.at[b], sem.at[b]),
          pltpu.make_async_copy(
              ei_hbm.at[g, 0, pl.ds(q * EQ, EQ)], src_v.at[b], sem.at[b]),
          pltpu.make_async_copy(
              ei_hbm.at[g, 1, pl.ds(q * EQ, EQ)], dst_v.at[b], sem.at[b]),
          pltpu.make_async_copy(
              ea_hbm.at[g, 0, pl.ds(q * EQ, EQ)], ea_v.at[b], sem.at[b]),
      )

    def start_in(t, b):
      for c in in_copies(t, b):
        c.start()

    def wait_in(t, b):
      for c in in_copies(t, b):
        c.wait()

    def compute(t, b):
      g = t // Q
      q = t % Q

      @plsc.parallel_loop(0, NC // L, unroll=8)
      def zloop(n):
        aggr_v[pl.ds(n * L, L)] = zero

      xb = x_v.at[b]
      sb = src_v.at[b]
      db = dst_v.at[b]
      ab = ea_v.at[b]

      @plsc.parallel_loop(0, N, unroll=4)
      def preadd(n):
        plsc.addupdate(xb.at[pl.ds(n * IN_C, L)], be0)
        plsc.addupdate(xb.at[pl.ds(n * IN_C + L, L)], be1)

      @plsc.parallel_loop(0, EQ // L, unroll=3)
      def eloop(j):
        base = j * L
        s16 = sb[pl.ds(base, L)] * IN_C
        d16 = db[pl.ds(base, L)] * IN_C
        a16 = ab[pl.ds(base, L)]
        for lane in range(L):
          s = pl.multiple_of(s16[lane], IN_C)
          d = pl.multiple_of(d16[lane], IN_C)
          a = a16[lane]
          m0 = jnp.maximum(xb[pl.ds(s, L)] + a * we0, 0.0)
          m1 = jnp.maximum(xb[pl.ds(s + L, L)] + a * we1, 0.0)
          plsc.addupdate(aggr_v.at[pl.ds(d, L)], m0)
          plsc.addupdate(aggr_v.at[pl.ds(d + L, L)], m1)

      pltpu.sync_copy(aggr_v, out_hbm.at[g, q])

    t0 = wid * TPW
    start_in(t0, 0)

    def pair(i, carry):
      ta = t0 + 2 * i
      start_in(ta + 1, 1)
      wait_in(ta, 0)
      compute(ta, 0)
      start_in(ta + 2, 0)
      wait_in(ta + 1, 1)
      compute(ta + 1, 1)
      return carry

    lax.fori_loop(0, (TPW - 1) // 2, pair, 0)
    tl = t0 + TPW - 1
    wait_in(tl, 0)
    compute(tl, 0)

  return k(x, ei, ea, params)


GB = 24


RPG = NC // 128


def _mlp_body(x_ref, p_ref, w1b_ref, b1t_ref, sel_ref, w2_ref, b2_ref,
              o_ref):
  p = p_ref[...]
  psum = (p[:, 0 * RPG:1 * RPG] + p[:, 1 * RPG:2 * RPG]
          + p[:, 2 * RPG:3 * RPG] + p[:, 3 * RPG:4 * RPG])
  h = x_ref[...] + psum
  a1 = jnp.maximum(
      h.reshape(GB * RPG, 128) @ w1b_ref[...] + b1t_ref[...], 0.0)
  s = a1.reshape(GB, RPG, 4 * HID).sum(axis=1)
  pooled = (s @ sel_ref[...]) * (1.0 / N)
  o_ref[...] = pooled @ w2_ref[...] + b2_ref[...]


def _mlp(x, partials, W1blk, b1t, sel, W2, b2):
  return pl.pallas_call(
      _mlp_body,
      out_shape=jax.ShapeDtypeStruct((G, OUT), jnp.float32),
      grid=(G // GB,),
      in_specs=[
          pl.BlockSpec((GB, RPG, 128), lambda i: (i, 0, 0)),
          pl.BlockSpec((GB, Q * RPG, 128), lambda i: (i, 0, 0)),
          pl.BlockSpec((128, 4 * HID), lambda i: (0, 0)),
          pl.BlockSpec((1, 4 * HID), lambda i: (0, 0)),
          pl.BlockSpec((4 * HID, HID), lambda i: (0, 0)),
          pl.BlockSpec((HID, OUT), lambda i: (0, 0)),
          pl.BlockSpec((1, OUT), lambda i: (0, 0)),
      ],
      out_specs=pl.BlockSpec((GB, OUT), lambda i: (i, 0)),
  )(x, partials, W1blk, b1t, sel, W2, b2)


def kernel(x, edge_index, edge_attr, W_edge, b_edge, W1, b1, W2, b2):
  x3 = x.reshape(G, N, IN_C)
  ei = edge_index.reshape(G, 2, E).astype(jnp.int32)
  ea = edge_attr.reshape(G, 1, E)
  params = jnp.concatenate(
      [W_edge.reshape(-1), b_edge.reshape(-1)]).astype(jnp.float32)
  partials = _sc_scatter(x3.reshape(G, 1, NC), ei, ea, params)
  W1blk = jnp.kron(jnp.eye(4, dtype=W1.dtype), W1)
  b1t = jnp.tile(b1, 4).reshape(1, 4 * HID)
  sel = jnp.tile(jnp.eye(HID, dtype=W1.dtype), (4, 1))
  tokens = _mlp(x3.reshape(G, RPG, 128),
                partials.reshape(G, Q * RPG, 128),
                W1blk, b1t, sel, W2, b2.reshape(1, OUT))
  return tokens.reshape(B, BANDS, OUT)
